# Initial kernel scaffold; baseline (speedup 1.0000x reference)
import jax
import jax.numpy as jnp
from jax.experimental import pallas as pl


def kernel(node_features, edge_index, W1, b1, W2, b2, Wc1, bc1, Wc2, bc2):
    raise NotImplementedError("write your pallas kernel here")


# speedup vs baseline: 17.2629x; 17.2629x over previous
import functools

import jax
import jax.numpy as jnp
from jax import lax
from jax.experimental import pallas as pl
from jax.experimental.pallas import tpu as pltpu
from jax.experimental.pallas import tpu_sc as plsc

N = 10000
D = 128
E = 320000

NC = 2
NS = 16
CH = 80
ST = (E // (NC * NS)) // CH
NP = 10240
RPT = NP // NS

_MESH = plsc.VectorSubcoreMesh(core_axis_name="c", subcore_axis_name="s")


@functools.partial(
    pl.kernel,
    mesh=_MESH,
    out_type=jax.ShapeDtypeStruct((NC, NP, D), jnp.float32),
    scratch_types=[
        pltpu.VMEM((ST, CH), jnp.int32),
        pltpu.VMEM((CH, D), jnp.float32),
        pltpu.VMEM_SHARED((NP, D), jnp.float32),
        pltpu.SemaphoreType.DMA,
    ],
)
def _sc_degree(dst_hbm, zeros_hbm, ones_hbm, out_hbm,
               dst_v, ones_v, acc, sem):
    c = lax.axis_index("c")
    s = lax.axis_index("s")
    r0 = s * RPT
    pltpu.sync_copy(dst_hbm.at[c, s], dst_v)
    pltpu.sync_copy(ones_hbm, ones_v)
    pltpu.sync_copy(zeros_hbm.at[pl.ds(r0, RPT)], acc.at[pl.ds(r0, RPT)])
    plsc.subcore_barrier()

    def step(j, carry):
        pltpu.sync_copy(ones_v, acc.at[dst_v.at[j]], add=True)
        return carry

    lax.fori_loop(0, ST, step, 0)
    plsc.subcore_barrier()
    pltpu.sync_copy(acc.at[pl.ds(r0, RPT)], out_hbm.at[c, pl.ds(r0, RPT)])


@functools.partial(
    pl.kernel,
    mesh=_MESH,
    out_type=jax.ShapeDtypeStruct((NC, NP, D), jnp.float32),
    scratch_types=[
        pltpu.VMEM((ST, CH), jnp.int32),
        pltpu.VMEM((ST, CH), jnp.int32),
        pltpu.VMEM((CH, D), jnp.float32),
        pltpu.VMEM_SHARED((NP, D), jnp.float32),
        pltpu.SemaphoreType.DMA,
    ],
)
def _sc_scatter(g_hbm, eidx_hbm, zeros_hbm, out_hbm,
                src_v, dst_v, rows_v, acc, sem):
    c = lax.axis_index("c")
    s = lax.axis_index("s")
    r0 = s * RPT
    pltpu.sync_copy(eidx_hbm.at[0, c, s], src_v)
    pltpu.sync_copy(eidx_hbm.at[1, c, s], dst_v)
    pltpu.sync_copy(zeros_hbm.at[pl.ds(r0, RPT)], acc.at[pl.ds(r0, RPT)])
    plsc.subcore_barrier()

    def step(j, carry):
        pltpu.async_copy(g_hbm.at[src_v.at[j]], rows_v, sem).wait()
        pltpu.sync_copy(rows_v, acc.at[dst_v.at[j]], add=True)
        return carry

    lax.fori_loop(0, ST, step, 0)
    plsc.subcore_barrier()
    pltpu.sync_copy(acc.at[pl.ds(r0, RPT)], out_hbm.at[c, pl.ds(r0, RPT)])


_BR = 400
_GRID = N // _BR


def _dinv_block(degw_ref, i):
    deg = degw_ref[0, :, 0:1] + degw_ref[1, :, 0:1] + 1.0
    return lax.rsqrt(deg)


def _tc_mlp_body(x_ref, w1_ref, b1_ref, w2_ref, b2_ref, wc1_ref, degw_ref,
                 g_ref):
    i = pl.program_id(0)
    h = jnp.dot(x_ref[...], w1_ref[...], preferred_element_type=jnp.float32)
    h = jnp.maximum(h + b1_ref[...], 0.0)
    h = jnp.dot(h, w2_ref[...], preferred_element_type=jnp.float32) + b2_ref[...]
    h = jnp.dot(h, wc1_ref[...], preferred_element_type=jnp.float32)
    g_ref[...] = h * _dinv_block(degw_ref, i)


def _tc_mid_body(p_ref, g_ref, degw_ref, bc1_ref, wc2_ref, out_ref):
    i = pl.program_id(0)
    dinv = _dinv_block(degw_ref, i)
    t = (p_ref[0] + p_ref[1] + g_ref[...]) * dinv + bc1_ref[...]
    t = jnp.maximum(t, 0.0)
    out_ref[...] = jnp.dot(t, wc2_ref[...],
                           preferred_element_type=jnp.float32) * dinv


def _tc_out_body(q_ref, g_ref, degw_ref, bc2_ref, out_ref):
    i = pl.program_id(0)
    dinv = _dinv_block(degw_ref, i)
    t = (q_ref[0] + q_ref[1] + g_ref[...]) * dinv + bc2_ref[...]
    out_ref[...] = jax.nn.sigmoid(t)


def _full(shape):
    return pl.BlockSpec(shape, lambda i: tuple(0 for _ in shape))


_ROWS = pl.BlockSpec((_BR, D), lambda i: (i, 0))
_PART = pl.BlockSpec((NC, _BR, D), lambda i: (0, i, 0))
_DEGW = pl.BlockSpec((NC, _BR, D), lambda i: (0, i, 0))


def _tc_mlp(x, W1, b1, W2, b2, Wc1, degw):
    return pl.pallas_call(
        _tc_mlp_body,
        grid=(_GRID,),
        in_specs=[_ROWS, _full((D, D)), _full((1, D)), _full((D, 2 * D)),
                  _full((1, 2 * D)), _full((2 * D, D)), _DEGW],
        out_specs=_ROWS,
        out_shape=jax.ShapeDtypeStruct((N, D), jnp.float32),
    )(x, W1, b1, W2, b2, Wc1, degw)


def _tc_mid(P, g1, degw, bc1, Wc2):
    return pl.pallas_call(
        _tc_mid_body,
        grid=(_GRID,),
        in_specs=[_PART, _ROWS, _DEGW, _full((1, D)), _full((D, D))],
        out_specs=_ROWS,
        out_shape=jax.ShapeDtypeStruct((N, D), jnp.float32),
    )(P, g1, degw, bc1, Wc2)


def _tc_out(Q, g2, degw, bc2):
    return pl.pallas_call(
        _tc_out_body,
        grid=(_GRID,),
        in_specs=[_PART, _ROWS, _DEGW, _full((1, D))],
        out_specs=_ROWS,
        out_shape=jax.ShapeDtypeStruct((N, D), jnp.float32),
    )(Q, g2, degw, bc2)


def kernel(node_features, edge_index, W1, b1, W2, b2, Wc1, bc1, Wc2, bc2):
    zeros128 = jnp.zeros((NP, D), jnp.float32)
    ones128 = jnp.ones((CH, D), jnp.float32)
    eidx = edge_index.reshape(2, NC, NS, ST, CH)

    degw = _sc_degree(eidx[1], zeros128, ones128)
    g1 = _tc_mlp(node_features, W1, b1.reshape(1, D), W2, b2.reshape(1, 2 * D),
                 Wc1, degw)
    P = _sc_scatter(g1, eidx, zeros128)
    g2 = _tc_mid(P, g1, degw, bc1.reshape(1, D), Wc2)
    Q = _sc_scatter(g2, eidx, zeros128)
    return _tc_out(Q, g2, degw, bc2.reshape(1, D))


# speedup vs baseline: 21.0617x; 1.2201x over previous
import functools

import jax
import jax.numpy as jnp
from jax import lax
from jax.experimental import pallas as pl
from jax.experimental.pallas import tpu as pltpu
from jax.experimental.pallas import tpu_sc as plsc

N = 10000
D = 128
E = 320000

NC = 2
NS = 16
CH = 128
ST = 80
EPT = ST * CH
NP = 10240
RPT = NP // NS

_MESH = plsc.VectorSubcoreMesh(core_axis_name="c", subcore_axis_name="s")


@functools.partial(
    pl.kernel,
    mesh=_MESH,
    out_type=jax.ShapeDtypeStruct((NC, NP, D), jnp.float32),
    scratch_types=[
        pltpu.VMEM((ST, CH), jnp.int32),
        pltpu.VMEM((CH, D), jnp.float32),
        pltpu.VMEM_SHARED((NP, D), jnp.float32),
        pltpu.SemaphoreType.DMA,
    ],
)
def _sc_degree(dst_hbm, zeros_hbm, ones_hbm, out_hbm,
               dst_v, ones_v, acc, sem):
    c = lax.axis_index("c")
    s = lax.axis_index("s")
    r0 = s * RPT
    pltpu.sync_copy(dst_hbm.at[c, s], dst_v)
    pltpu.sync_copy(ones_hbm, ones_v)
    pltpu.sync_copy(zeros_hbm.at[pl.ds(r0, RPT)], acc.at[pl.ds(r0, RPT)])
    plsc.subcore_barrier()

    def step(j, carry):
        pltpu.sync_copy(ones_v, acc.at[dst_v.at[j]], add=True)
        return carry

    lax.fori_loop(0, ST, step, 0)
    plsc.subcore_barrier()
    pltpu.sync_copy(acc.at[pl.ds(r0, RPT)], out_hbm.at[c, pl.ds(r0, RPT)])


@functools.partial(
    pl.kernel,
    mesh=_MESH,
    out_type=jax.ShapeDtypeStruct((NC, NP, D), jnp.float32),
    scratch_types=[
        pltpu.VMEM((CH,), jnp.int32),
        pltpu.VMEM((CH,), jnp.int32),
        pltpu.VMEM((CH,), jnp.int32),
        pltpu.VMEM((CH,), jnp.int32),
        pltpu.VMEM((CH, D), jnp.float32),
        pltpu.VMEM((CH, D), jnp.float32),
        pltpu.VMEM_SHARED((NP, D), jnp.float32),
        pltpu.SemaphoreType.DMA,
        pltpu.SemaphoreType.DMA,
    ],
)
def _sc_scatter(g_hbm, eidx_hbm, zeros_hbm, out_hbm,
                src0_v, src1_v, dst0_v, dst1_v, rows0_v, rows1_v, acc,
                gsem0, gsem1):
    c = lax.axis_index("c")
    s = lax.axis_index("s")
    r0 = s * RPT
    pltpu.sync_copy(zeros_hbm.at[pl.ds(r0, RPT)], acc.at[pl.ds(r0, RPT)])
    plsc.subcore_barrier()

    pltpu.sync_copy(eidx_hbm.at[0, c, s, 0], src0_v)
    pltpu.sync_copy(eidx_hbm.at[1, c, s, 0], dst0_v)
    pltpu.async_copy(g_hbm.at[src0_v], rows0_v, gsem0)
    pltpu.sync_copy(eidx_hbm.at[0, c, s, 1], src1_v)
    pltpu.sync_copy(eidx_hbm.at[1, c, s, 1], dst1_v)
    pltpu.async_copy(g_hbm.at[src1_v], rows1_v, gsem1)

    def step(i, carry):
        j = 2 * i
        pltpu.make_async_copy(g_hbm.at[src0_v], rows0_v, gsem0).wait()
        pltpu.sync_copy(rows0_v, acc.at[dst0_v], add=True)

        @pl.when(j + 2 < ST)
        def _():
            pltpu.sync_copy(eidx_hbm.at[0, c, s, j + 2], src0_v)
            pltpu.sync_copy(eidx_hbm.at[1, c, s, j + 2], dst0_v)
            pltpu.async_copy(g_hbm.at[src0_v], rows0_v, gsem0)

        pltpu.make_async_copy(g_hbm.at[src1_v], rows1_v, gsem1).wait()
        pltpu.sync_copy(rows1_v, acc.at[dst1_v], add=True)

        @pl.when(j + 3 < ST)
        def _():
            pltpu.sync_copy(eidx_hbm.at[0, c, s, j + 3], src1_v)
            pltpu.sync_copy(eidx_hbm.at[1, c, s, j + 3], dst1_v)
            pltpu.async_copy(g_hbm.at[src1_v], rows1_v, gsem1)

        return carry

    lax.fori_loop(0, ST // 2, step, 0)
    plsc.subcore_barrier()
    pltpu.sync_copy(acc.at[pl.ds(r0, RPT)], out_hbm.at[c, pl.ds(r0, RPT)])


_BR = 400
_GRID = N // _BR


def _dinv_block(degw_ref, i):
    deg = degw_ref[0, :, 0:1] + degw_ref[1, :, 0:1] + 1.0
    return lax.rsqrt(deg)


def _tc_mlp_body(x_ref, w1_ref, b1_ref, w2_ref, b2_ref, wc1_ref, degw_ref,
                 g_ref):
    i = pl.program_id(0)
    h = jnp.dot(x_ref[...], w1_ref[...], preferred_element_type=jnp.float32)
    h = jnp.maximum(h + b1_ref[...], 0.0)
    h = jnp.dot(h, w2_ref[...], preferred_element_type=jnp.float32) + b2_ref[...]
    h = jnp.dot(h, wc1_ref[...], preferred_element_type=jnp.float32)
    g_ref[...] = h * _dinv_block(degw_ref, i)


def _tc_mid_body(p_ref, g_ref, degw_ref, bc1_ref, wc2_ref, out_ref):
    i = pl.program_id(0)
    dinv = _dinv_block(degw_ref, i)
    t = (p_ref[0] + p_ref[1] + g_ref[...]) * dinv + bc1_ref[...]
    t = jnp.maximum(t, 0.0)
    out_ref[...] = jnp.dot(t, wc2_ref[...],
                           preferred_element_type=jnp.float32) * dinv


def _tc_out_body(q_ref, g_ref, degw_ref, bc2_ref, out_ref):
    i = pl.program_id(0)
    dinv = _dinv_block(degw_ref, i)
    t = (q_ref[0] + q_ref[1] + g_ref[...]) * dinv + bc2_ref[...]
    out_ref[...] = jax.nn.sigmoid(t)


def _full(shape):
    return pl.BlockSpec(shape, lambda i: tuple(0 for _ in shape))


_ROWS = pl.BlockSpec((_BR, D), lambda i: (i, 0))
_PART = pl.BlockSpec((NC, _BR, D), lambda i: (0, i, 0))
_DEGW = pl.BlockSpec((NC, _BR, D), lambda i: (0, i, 0))


def _tc_mlp(x, W1, b1, W2, b2, Wc1, degw):
    return pl.pallas_call(
        _tc_mlp_body,
        grid=(_GRID,),
        in_specs=[_ROWS, _full((D, D)), _full((1, D)), _full((D, 2 * D)),
                  _full((1, 2 * D)), _full((2 * D, D)), _DEGW],
        out_specs=_ROWS,
        out_shape=jax.ShapeDtypeStruct((N, D), jnp.float32),
    )(x, W1, b1, W2, b2, Wc1, degw)


def _tc_mid(P, g1, degw, bc1, Wc2):
    return pl.pallas_call(
        _tc_mid_body,
        grid=(_GRID,),
        in_specs=[_PART, _ROWS, _DEGW, _full((1, D)), _full((D, D))],
        out_specs=_ROWS,
        out_shape=jax.ShapeDtypeStruct((N, D), jnp.float32),
    )(P, g1, degw, bc1, Wc2)


def _tc_out(Q, g2, degw, bc2):
    return pl.pallas_call(
        _tc_out_body,
        grid=(_GRID,),
        in_specs=[_PART, _ROWS, _DEGW, _full((1, D))],
        out_specs=_ROWS,
        out_shape=jax.ShapeDtypeStruct((N, D), jnp.float32),
    )(Q, g2, degw, bc2)


def kernel(node_features, edge_index, W1, b1, W2, b2, Wc1, bc1, Wc2, bc2):
    zeros128 = jnp.zeros((NP, D), jnp.float32)
    ones128 = jnp.ones((CH, D), jnp.float32)
    rowpad = jnp.zeros((NP - N, D), jnp.float32)

    ept = E // (NC * NS)
    er = edge_index.reshape(2, NC, NS, ept)
    pads = jnp.broadcast_to(
        jnp.arange(N, NP, dtype=edge_index.dtype), (2, NC, NS, EPT - ept))
    eidx = jnp.concatenate([er, pads], axis=-1).reshape(2, NC, NS, ST, CH)

    degw = _sc_degree(eidx[1], zeros128, ones128)
    g1 = _tc_mlp(node_features, W1, b1.reshape(1, D), W2, b2.reshape(1, 2 * D),
                 Wc1, degw)
    P = _sc_scatter(jnp.concatenate([g1, rowpad]), eidx, zeros128)
    g2 = _tc_mid(P, g1, degw, bc1.reshape(1, D), Wc2)
    Q = _sc_scatter(jnp.concatenate([g2, rowpad]), eidx, zeros128)
    return _tc_out(Q, g2, degw, bc2.reshape(1, D))


# speedup vs baseline: 21.0795x; 1.0008x over previous
import functools

import jax
import jax.numpy as jnp
from jax import lax
from jax.experimental import pallas as pl
from jax.experimental.pallas import tpu as pltpu
from jax.experimental.pallas import tpu_sc as plsc

N = 10000
D = 128
E = 320000

NC = 2
NS = 16
CH = 128
ST = 80
EPT = ST * CH
NP = 10240
RPT = NP // NS

_MESH = plsc.VectorSubcoreMesh(core_axis_name="c", subcore_axis_name="s")


@functools.partial(
    pl.kernel,
    mesh=_MESH,
    out_type=jax.ShapeDtypeStruct((NC, NP, D), jnp.float32),
    scratch_types=[
        pltpu.VMEM((ST, CH), jnp.int32),
        pltpu.VMEM((CH, D), jnp.float32),
        pltpu.VMEM_SHARED((NP, D), jnp.float32),
        pltpu.SemaphoreType.DMA,
    ],
)
def _sc_degree(dst_hbm, zeros_hbm, ones_hbm, out_hbm,
               dst_v, ones_v, acc, sem):
    c = lax.axis_index("c")
    s = lax.axis_index("s")
    r0 = s * RPT
    pltpu.sync_copy(dst_hbm.at[c, s], dst_v)
    pltpu.sync_copy(ones_hbm, ones_v)
    pltpu.sync_copy(zeros_hbm.at[pl.ds(r0, RPT)], acc.at[pl.ds(r0, RPT)])
    plsc.subcore_barrier()

    def step(j, carry):
        pltpu.sync_copy(ones_v, acc.at[dst_v.at[j]], add=True)
        return carry

    lax.fori_loop(0, ST, step, 0)
    plsc.subcore_barrier()
    pltpu.sync_copy(acc.at[pl.ds(r0, RPT)], out_hbm.at[c, pl.ds(r0, RPT)])


@functools.partial(
    pl.kernel,
    mesh=_MESH,
    out_type=jax.ShapeDtypeStruct((NC, NP, D), jnp.float32),
    scratch_types=[
        pltpu.VMEM((CH,), jnp.int32),
        pltpu.VMEM((CH,), jnp.int32),
        pltpu.VMEM((CH,), jnp.int32),
        pltpu.VMEM((CH,), jnp.int32),
        pltpu.VMEM((CH, D), jnp.float32),
        pltpu.VMEM((CH, D), jnp.float32),
        pltpu.VMEM_SHARED((NP, D), jnp.float32),
        pltpu.SemaphoreType.DMA,
        pltpu.SemaphoreType.DMA,
    ],
)
def _sc_scatter(g_hbm, eidx_hbm, zeros_hbm, out_hbm,
                src0_v, src1_v, dst0_v, dst1_v, rows0_v, rows1_v, acc,
                gsem0, gsem1):
    c = lax.axis_index("c")
    s = lax.axis_index("s")
    r0 = s * RPT
    pltpu.sync_copy(zeros_hbm.at[pl.ds(r0, RPT)], acc.at[pl.ds(r0, RPT)])
    plsc.subcore_barrier()

    pltpu.sync_copy(eidx_hbm.at[0, c, s, 0], src0_v)
    pltpu.sync_copy(eidx_hbm.at[1, c, s, 0], dst0_v)
    pltpu.async_copy(g_hbm.at[src0_v], rows0_v, gsem0)
    pltpu.sync_copy(eidx_hbm.at[0, c, s, 1], src1_v)
    pltpu.sync_copy(eidx_hbm.at[1, c, s, 1], dst1_v)
    pltpu.async_copy(g_hbm.at[src1_v], rows1_v, gsem1)

    def step(i, carry):
        j = 2 * i
        pltpu.make_async_copy(g_hbm.at[src0_v], rows0_v, gsem0).wait()
        pltpu.sync_copy(rows0_v, acc.at[dst0_v], add=True)

        @pl.when(j + 2 < ST)
        def _():
            pltpu.sync_copy(eidx_hbm.at[0, c, s, j + 2], src0_v)
            pltpu.sync_copy(eidx_hbm.at[1, c, s, j + 2], dst0_v)
            pltpu.async_copy(g_hbm.at[src0_v], rows0_v, gsem0)

        pltpu.make_async_copy(g_hbm.at[src1_v], rows1_v, gsem1).wait()
        pltpu.sync_copy(rows1_v, acc.at[dst1_v], add=True)

        @pl.when(j + 3 < ST)
        def _():
            pltpu.sync_copy(eidx_hbm.at[0, c, s, j + 3], src1_v)
            pltpu.sync_copy(eidx_hbm.at[1, c, s, j + 3], dst1_v)
            pltpu.async_copy(g_hbm.at[src1_v], rows1_v, gsem1)

        return carry

    lax.fori_loop(0, ST // 2, step, 0)
    plsc.subcore_barrier()
    pltpu.sync_copy(acc.at[pl.ds(r0, RPT)], out_hbm.at[c, pl.ds(r0, RPT)])


_BR = 400
_GRID = N // _BR
_BRP = 320
_GRIDP = NP // _BRP


def _dinv_block(degw_ref):
    deg = degw_ref[0, :, 0:1] + degw_ref[1, :, 0:1] + 1.0
    return lax.rsqrt(deg)


def _tc_mlp_body(x_ref, w1_ref, b1_ref, w2_ref, b2_ref, wc1_ref, degw_ref,
                 g_ref):
    h = jnp.dot(x_ref[...], w1_ref[...], preferred_element_type=jnp.float32)
    h = jnp.maximum(h + b1_ref[...], 0.0)
    h = jnp.dot(h, w2_ref[...], preferred_element_type=jnp.float32) + b2_ref[...]
    h = jnp.dot(h, wc1_ref[...], preferred_element_type=jnp.float32)
    g_ref[...] = h * _dinv_block(degw_ref)


def _tc_mid_body(p_ref, g_ref, degw_ref, bc1_ref, wc2_ref, out_ref):
    dinv = _dinv_block(degw_ref)
    t = (p_ref[0] + p_ref[1] + g_ref[...]) * dinv + bc1_ref[...]
    t = jnp.maximum(t, 0.0)
    out_ref[...] = jnp.dot(t, wc2_ref[...],
                           preferred_element_type=jnp.float32) * dinv


def _tc_out_body(q_ref, g_ref, degw_ref, bc2_ref, out_ref):
    dinv = _dinv_block(degw_ref)
    t = (q_ref[0] + q_ref[1] + g_ref[...]) * dinv + bc2_ref[...]
    out_ref[...] = jax.nn.sigmoid(t)


def _full(shape):
    return pl.BlockSpec(shape, lambda i: tuple(0 for _ in shape))


_ROWS = pl.BlockSpec((_BR, D), lambda i: (i, 0))
_PART = pl.BlockSpec((NC, _BR, D), lambda i: (0, i, 0))
_DEGW = pl.BlockSpec((NC, _BR, D), lambda i: (0, i, 0))
_ROWSP = pl.BlockSpec((_BRP, D), lambda i: (i, 0))
_PARTP = pl.BlockSpec((NC, _BRP, D), lambda i: (0, i, 0))
_DEGWP = pl.BlockSpec((NC, _BRP, D), lambda i: (0, i, 0))


def _tc_mlp(x, W1, b1, W2, b2, Wc1, degw):
    return pl.pallas_call(
        _tc_mlp_body,
        grid=(_GRIDP,),
        in_specs=[_ROWSP, _full((D, D)), _full((1, D)), _full((D, 2 * D)),
                  _full((1, 2 * D)), _full((2 * D, D)), _DEGWP],
        out_specs=_ROWSP,
        out_shape=jax.ShapeDtypeStruct((NP, D), jnp.float32),
    )(x, W1, b1, W2, b2, Wc1, degw)


def _tc_mid(P, g1, degw, bc1, Wc2):
    return pl.pallas_call(
        _tc_mid_body,
        grid=(_GRIDP,),
        in_specs=[_PARTP, _ROWSP, _DEGWP, _full((1, D)), _full((D, D))],
        out_specs=_ROWSP,
        out_shape=jax.ShapeDtypeStruct((NP, D), jnp.float32),
    )(P, g1, degw, bc1, Wc2)


def _tc_out(Q, g2, degw, bc2):
    return pl.pallas_call(
        _tc_out_body,
        grid=(_GRID,),
        in_specs=[_PART, _ROWS, _DEGW, _full((1, D))],
        out_specs=_ROWS,
        out_shape=jax.ShapeDtypeStruct((N, D), jnp.float32),
    )(Q, g2, degw, bc2)


def kernel(node_features, edge_index, W1, b1, W2, b2, Wc1, bc1, Wc2, bc2):
    zeros128 = jnp.zeros((NP, D), jnp.float32)
    ones128 = jnp.ones((CH, D), jnp.float32)

    ept = E // (NC * NS)
    er = edge_index.reshape(2, NC, NS, ept)
    pads = jnp.broadcast_to(
        jnp.arange(N, NP, dtype=edge_index.dtype), (2, NC, NS, EPT - ept))
    eidx = jnp.concatenate([er, pads], axis=-1).reshape(2, NC, NS, ST, CH)

    degw = _sc_degree(eidx[1], zeros128, ones128)
    g1 = _tc_mlp(node_features, W1, b1.reshape(1, D), W2, b2.reshape(1, 2 * D),
                 Wc1, degw)
    P = _sc_scatter(g1, eidx, zeros128)
    g2 = _tc_mid(P, g1, degw, bc1.reshape(1, D), Wc2)
    Q = _sc_scatter(g2, eidx, zeros128)
    return _tc_out(Q, g2, degw, bc2.reshape(1, D))
